# baseline (speedup 1.0000x reference)
import numpy as np
import jax
import jax.numpy as jnp
from jax import lax
from jax.experimental import pallas as pl
from jax.experimental.pallas import tpu as pltpu

_FX = 518.86
_FY = 519.47
_U0 = 272.0
_V0 = 208.0
_H, _W = 416, 544
_B = 4
_EPS = 1e-4
_N = _H * _W
_K1 = _N // 2
_K2 = _N // 2 + 1
_HI0 = 0x43800000
_UPPER_MULT = float(np.float32(1.33))


def _gauss5():
    sigma = 1.1
    xs = np.arange(5, dtype=np.float64) - 2.0
    g = np.exp(-(xs ** 2) / (2.0 * sigma ** 2)).astype(np.float32)
    g = g / g.sum()
    return [float(v) for v in g]


_G = _gauss5()


def _vb_kernel(rgb_ref, pd_ref, gt_ref, out_ref, bits_ref):
    H, W, B = _H, _W, _B

    gray = (0.114 * rgb_ref[:, 0] + 0.587 * rgb_ref[:, 1]
            + 0.299 * rgb_ref[:, 2])
    gray = jnp.floor(jnp.clip(gray, 0.0, 255.0))

    pc = jnp.pad(gray, ((0, 0), (0, 0), (2, 2)))
    t = (_G[0] * pc[:, :, 0:W] + _G[1] * pc[:, :, 1:W + 1]
         + _G[2] * pc[:, :, 2:W + 2] + _G[3] * pc[:, :, 3:W + 3]
         + _G[4] * pc[:, :, 4:W + 4])
    pr = jnp.pad(t, ((0, 0), (2, 2), (0, 0)))
    blur = (_G[0] * pr[:, 0:H] + _G[1] * pr[:, 1:H + 1]
            + _G[2] * pr[:, 2:H + 2] + _G[3] * pr[:, 3:H + 3]
            + _G[4] * pr[:, 4:H + 4])
    bits_ref[:, :, :] = lax.bitcast_convert_type(blur, jnp.int32)

    def search_body(_, st):
        los, his = st
        nlos, nhis = [], []
        for b in range(B):
            mid = los[b] + (his[b] - los[b]) // 2
            c = jnp.sum((bits_ref[b] <= mid).astype(jnp.float32))
            take = c >= _K1
            nlos.append(jnp.where(take, los[b], mid + 1))
            nhis.append(jnp.where(take, mid, his[b]))
        return tuple(nlos), tuple(nhis)

    z = jnp.int32(0)
    hi0 = jnp.int32(_HI0)
    _, his_t = lax.fori_loop(0, 31, search_body, ((z,) * B, (hi0,) * B))
    his = jnp.stack(his_t)

    bits = bits_ref[:, :, :]
    blur = lax.bitcast_convert_type(bits, jnp.float32)
    le = bits <= his[:, None, None]
    c1 = jnp.sum(le.astype(jnp.float32), axis=(1, 2))
    v1 = jnp.max(jnp.where(le, blur, -jnp.inf), axis=(1, 2))
    v2 = jnp.min(jnp.where(le, jnp.inf, blur), axis=(1, 2))
    v2 = jnp.where(c1 >= _K2, v1, v2)
    med = (v1 + v2) * 0.5
    upper = jnp.minimum(255.0, jnp.floor(_UPPER_MULT * med))

    pb = jnp.pad(blur, ((0, 0), (0, 0), (1, 1)))
    dx = pb[:, :, 2:W + 2] - pb[:, :, 0:W]
    sm = pb[:, :, 0:W] + 2.0 * pb[:, :, 1:W + 1] + pb[:, :, 2:W + 2]
    pdx = jnp.pad(dx, ((0, 0), (1, 1), (0, 0)))
    gx = pdx[:, 0:H] + 2.0 * pdx[:, 1:H + 1] + pdx[:, 2:H + 2]
    psm = jnp.pad(sm, ((0, 0), (1, 1), (0, 0)))
    gy = psm[:, 2:H + 2] - psm[:, 0:H]
    mag = jnp.sqrt(gx * gx + gy * gy + 1e-12)
    edge = (mag > upper[:, None, None]).astype(jnp.float32)

    p = jnp.pad(edge, ((0, 0), (0, 0), (2, 2)))
    m5 = p[:, :, 0:W]
    for j in range(1, 5):
        m5 = jnp.maximum(m5, p[:, :, j:j + W])
    p2 = jnp.pad(m5, ((0, 0), (2, 2), (0, 0)))
    d5 = p2[:, 0:H]
    for i in range(1, 5):
        d5 = jnp.maximum(d5, p2[:, i:i + H])
    bg = d5 < 1.0

    gt = gt_ref[:, 0] / 10.0
    pd = pd_ref[:, 0] / 10.0
    pd = jnp.where(pd < 0.0, 0.001, pd)
    col = lax.broadcasted_iota(jnp.int32, (H, W), 1).astype(jnp.float32)
    row = lax.broadcasted_iota(jnp.int32, (H, W), 0).astype(jnp.float32)
    uu = col - _U0
    vv = row - _V0
    r2c = (uu * uu + vv * vv)[None]
    r = gt / pd
    omr = 1.0 - r
    l1 = gt - pd
    dist = jnp.sqrt(r2c * (omr * omr) + l1 * l1 + _EPS)
    m = (gt > 0.0) & (gt <= 10.0) & bg
    mf = m.astype(jnp.float32)
    s_tot = jnp.sum(dist * mf)
    c_tot = jnp.sum(mf)
    out_ref[0, 0] = s_tot / jnp.maximum(c_tot, 1.0) / _FX


def kernel(rgb, depth_pred, depth_gt):
    out = pl.pallas_call(
        _vb_kernel,
        in_specs=[
            pl.BlockSpec((_B, 3, _H, _W), lambda: (0, 0, 0, 0)),
            pl.BlockSpec((_B, 1, _H, _W), lambda: (0, 0, 0, 0)),
            pl.BlockSpec((_B, 1, _H, _W), lambda: (0, 0, 0, 0)),
        ],
        out_specs=pl.BlockSpec((1, 1), lambda: (0, 0),
                               memory_space=pltpu.SMEM),
        out_shape=jax.ShapeDtypeStruct((1, 1), jnp.float32),
        scratch_shapes=[pltpu.VMEM((_B, _H, _W), jnp.int32)],
    )(rgb, depth_pred, depth_gt)
    return out[0, 0]


# speedup vs baseline: 1.1499x; 1.1499x over previous
import numpy as np
import jax
import jax.numpy as jnp
from jax import lax
from jax.experimental import pallas as pl
from jax.experimental.pallas import tpu as pltpu

_FX = 518.86
_FY = 519.47
_U0 = 272.0
_V0 = 208.0
_H, _W = 416, 544
_B = 4
_EPS = 1e-4
_N = _H * _W
_K1 = _N // 2
_K2 = _N // 2 + 1
_HI0 = 0x43800000
_UPPER_MULT = float(np.float32(1.33))


def _gauss5():
    sigma = 1.1
    xs = np.arange(5, dtype=np.float64) - 2.0
    g = np.exp(-(xs ** 2) / (2.0 * sigma ** 2)).astype(np.float32)
    g = g / g.sum()
    return [float(v) for v in g]


_G = _gauss5()


def _vb_kernel(rgb_ref, pd_ref, gt_ref, out_ref, bits_ref):
    H, W, B = _H, _W, _B

    gray = (0.114 * rgb_ref[:, 0] + 0.587 * rgb_ref[:, 1]
            + 0.299 * rgb_ref[:, 2])
    gray = jnp.floor(jnp.clip(gray, 0.0, 255.0))

    pc = jnp.pad(gray, ((0, 0), (0, 0), (2, 2)))
    t = (_G[0] * pc[:, :, 0:W] + _G[1] * pc[:, :, 1:W + 1]
         + _G[2] * pc[:, :, 2:W + 2] + _G[3] * pc[:, :, 3:W + 3]
         + _G[4] * pc[:, :, 4:W + 4])
    pr = jnp.pad(t, ((0, 0), (2, 2), (0, 0)))
    blur = (_G[0] * pr[:, 0:H] + _G[1] * pr[:, 1:H + 1]
            + _G[2] * pr[:, 2:H + 2] + _G[3] * pr[:, 3:H + 3]
            + _G[4] * pr[:, 4:H + 4])
    bits_ref[:, :, :] = lax.bitcast_convert_type(blur, jnp.int32)

    def search_body(_, st):
        los, his = st
        mid = los + (his - los) // 2
        mask = (bits_ref[:, :, :] <= mid[:, None, None]).astype(jnp.float32)
        c = jnp.sum(mask, axis=(1, 2))
        take = c >= _K1
        return (jnp.where(take, los, mid + 1), jnp.where(take, mid, his))

    los0 = jnp.zeros((B,), jnp.int32)
    his0 = jnp.full((B,), _HI0, jnp.int32)
    _, his = lax.fori_loop(0, 31, search_body, (los0, his0))

    bits = bits_ref[:, :, :]
    blur = lax.bitcast_convert_type(bits, jnp.float32)
    le = bits <= his[:, None, None]
    c1 = jnp.sum(le.astype(jnp.float32), axis=(1, 2))
    v1 = jnp.max(jnp.where(le, blur, -jnp.inf), axis=(1, 2))
    v2 = jnp.min(jnp.where(le, jnp.inf, blur), axis=(1, 2))
    v2 = jnp.where(c1 >= _K2, v1, v2)
    med = (v1 + v2) * 0.5
    upper = jnp.minimum(255.0, jnp.floor(_UPPER_MULT * med))

    pb = jnp.pad(blur, ((0, 0), (0, 0), (1, 1)))
    dx = pb[:, :, 2:W + 2] - pb[:, :, 0:W]
    sm = pb[:, :, 0:W] + 2.0 * pb[:, :, 1:W + 1] + pb[:, :, 2:W + 2]
    pdx = jnp.pad(dx, ((0, 0), (1, 1), (0, 0)))
    gx = pdx[:, 0:H] + 2.0 * pdx[:, 1:H + 1] + pdx[:, 2:H + 2]
    psm = jnp.pad(sm, ((0, 0), (1, 1), (0, 0)))
    gy = psm[:, 2:H + 2] - psm[:, 0:H]
    mag = jnp.sqrt(gx * gx + gy * gy + 1e-12)
    edge = (mag > upper[:, None, None]).astype(jnp.bfloat16)

    iw = lax.broadcasted_iota(jnp.int32, (W, W), 0)
    jw = lax.broadcasted_iota(jnp.int32, (W, W), 1)
    band_w = (jnp.abs(iw - jw) <= 2).astype(jnp.bfloat16)
    ih = lax.broadcasted_iota(jnp.int32, (H, H), 0)
    jh = lax.broadcasted_iota(jnp.int32, (H, H), 1)
    band_h = (jnp.abs(ih - jh) <= 2).astype(jnp.bfloat16)
    bgs = []
    for b in range(B):
        cs = lax.dot_general(edge[b], band_w, (((1,), (0,)), ((), ())),
                             preferred_element_type=jnp.float32)
        cs_bf = cs.astype(jnp.bfloat16)
        rs = lax.dot_general(band_h, cs_bf, (((1,), (0,)), ((), ())),
                             preferred_element_type=jnp.float32)
        bgs.append(rs)
    bg = jnp.stack(bgs) < 0.5

    gt = gt_ref[:, 0] / 10.0
    pd = pd_ref[:, 0] / 10.0
    pd = jnp.where(pd < 0.0, 0.001, pd)
    col = lax.broadcasted_iota(jnp.int32, (H, W), 1).astype(jnp.float32)
    row = lax.broadcasted_iota(jnp.int32, (H, W), 0).astype(jnp.float32)
    uu = col - _U0
    vv = row - _V0
    r2c = (uu * uu + vv * vv)[None]
    r = gt / pd
    omr = 1.0 - r
    l1 = gt - pd
    dist = jnp.sqrt(r2c * (omr * omr) + l1 * l1 + _EPS)
    m = (gt > 0.0) & (gt <= 10.0) & bg
    mf = m.astype(jnp.float32)
    s_tot = jnp.sum(dist * mf)
    c_tot = jnp.sum(mf)
    out_ref[0, 0] = s_tot / jnp.maximum(c_tot, 1.0) / _FX


def kernel(rgb, depth_pred, depth_gt):
    out = pl.pallas_call(
        _vb_kernel,
        in_specs=[
            pl.BlockSpec((_B, 3, _H, _W), lambda: (0, 0, 0, 0)),
            pl.BlockSpec((_B, 1, _H, _W), lambda: (0, 0, 0, 0)),
            pl.BlockSpec((_B, 1, _H, _W), lambda: (0, 0, 0, 0)),
        ],
        out_specs=pl.BlockSpec((1, 1), lambda: (0, 0),
                               memory_space=pltpu.SMEM),
        out_shape=jax.ShapeDtypeStruct((1, 1), jnp.float32),
        scratch_shapes=[pltpu.VMEM((_B, _H, _W), jnp.int32)],
    )(rgb, depth_pred, depth_gt)
    return out[0, 0]


# speedup vs baseline: 1.2084x; 1.0508x over previous
import numpy as np
import jax
import jax.numpy as jnp
from jax import lax
from jax.experimental import pallas as pl
from jax.experimental.pallas import tpu as pltpu

_FX = 518.86
_FY = 519.47
_U0 = 272.0
_V0 = 208.0
_H, _W = 416, 544
_B = 4
_EPS = 1e-4
_N = _H * _W
_K1 = _N // 2
_K2 = _N // 2 + 1
_HI0 = 0x43800000
_UPPER_MULT = float(np.float32(1.33))


def _gauss5():
    sigma = 1.1
    xs = np.arange(5, dtype=np.float64) - 2.0
    g = np.exp(-(xs ** 2) / (2.0 * sigma ** 2)).astype(np.float32)
    g = g / g.sum()
    return [float(v) for v in g]


_G = _gauss5()


def _vb_kernel(rgb_ref, pd_ref, gt_ref, out_ref, bits_ref):
    H, W, B = _H, _W, _B

    gray = (0.114 * rgb_ref[:, 0] + 0.587 * rgb_ref[:, 1]
            + 0.299 * rgb_ref[:, 2])
    gray = jnp.floor(jnp.clip(gray, 0.0, 255.0))

    iw = lax.broadcasted_iota(jnp.int32, (W, W), 0)
    jw = lax.broadcasted_iota(jnp.int32, (W, W), 1)
    adw = jnp.abs(iw - jw)
    gwf = jnp.where(adw == 0, _G[2],
                    jnp.where(adw == 1, _G[1],
                              jnp.where(adw == 2, _G[0], 0.0)))
    gw_hi = gwf.astype(jnp.bfloat16)
    gw_lo = (gwf - gw_hi.astype(jnp.float32)).astype(jnp.bfloat16)
    gray_bf = gray.astype(jnp.bfloat16)
    dn = (((1,), (0,)), ((), ()))
    t = jnp.stack([
        lax.dot_general(gray_bf[b], gw_hi, dn,
                        preferred_element_type=jnp.float32)
        + lax.dot_general(gray_bf[b], gw_lo, dn,
                          preferred_element_type=jnp.float32)
        for b in range(B)])
    pr = jnp.pad(t, ((0, 0), (2, 2), (0, 0)))
    blur = (_G[0] * pr[:, 0:H] + _G[1] * pr[:, 1:H + 1]
            + _G[2] * pr[:, 2:H + 2] + _G[3] * pr[:, 3:H + 3]
            + _G[4] * pr[:, 4:H + 4])
    bits_ref[:, :, :] = lax.bitcast_convert_type(blur, jnp.int32)

    def search_body(_, st):
        los, his = st
        mid = los + (his - los) // 2
        mask = (bits_ref[:, :, :] <= mid[:, None, None]).astype(jnp.float32)
        c = jnp.sum(mask, axis=(1, 2))
        take = c >= _K1
        return (jnp.where(take, los, mid + 1), jnp.where(take, mid, his))

    los0 = jnp.zeros((B,), jnp.int32)
    his0 = jnp.full((B,), _HI0, jnp.int32)
    _, his = lax.fori_loop(0, 31, search_body, (los0, his0))

    bits = bits_ref[:, :, :]
    blur = lax.bitcast_convert_type(bits, jnp.float32)
    le = bits <= his[:, None, None]
    c1 = jnp.sum(le.astype(jnp.float32), axis=(1, 2))
    v1 = jnp.max(jnp.where(le, blur, -jnp.inf), axis=(1, 2))
    v2 = jnp.min(jnp.where(le, jnp.inf, blur), axis=(1, 2))
    v2 = jnp.where(c1 >= _K2, v1, v2)
    med = (v1 + v2) * 0.5
    upper = jnp.minimum(255.0, jnp.floor(_UPPER_MULT * med))

    pb = jnp.pad(blur, ((0, 0), (0, 0), (1, 1)))
    dx = pb[:, :, 2:W + 2] - pb[:, :, 0:W]
    sm = pb[:, :, 0:W] + 2.0 * pb[:, :, 1:W + 1] + pb[:, :, 2:W + 2]
    pdx = jnp.pad(dx, ((0, 0), (1, 1), (0, 0)))
    gx = pdx[:, 0:H] + 2.0 * pdx[:, 1:H + 1] + pdx[:, 2:H + 2]
    psm = jnp.pad(sm, ((0, 0), (1, 1), (0, 0)))
    gy = psm[:, 2:H + 2] - psm[:, 0:H]
    mag = jnp.sqrt(gx * gx + gy * gy + 1e-12)
    edge = (mag > upper[:, None, None]).astype(jnp.bfloat16)

    iw = lax.broadcasted_iota(jnp.int32, (W, W), 0)
    jw = lax.broadcasted_iota(jnp.int32, (W, W), 1)
    band_w = (jnp.abs(iw - jw) <= 2).astype(jnp.bfloat16)
    ih = lax.broadcasted_iota(jnp.int32, (H, H), 0)
    jh = lax.broadcasted_iota(jnp.int32, (H, H), 1)
    band_h = (jnp.abs(ih - jh) <= 2).astype(jnp.bfloat16)
    bgs = []
    for b in range(B):
        cs = lax.dot_general(edge[b], band_w, (((1,), (0,)), ((), ())),
                             preferred_element_type=jnp.float32)
        cs_bf = cs.astype(jnp.bfloat16)
        rs = lax.dot_general(band_h, cs_bf, (((1,), (0,)), ((), ())),
                             preferred_element_type=jnp.float32)
        bgs.append(rs)
    bg = jnp.stack(bgs) < 0.5

    gt = gt_ref[:, 0] / 10.0
    pd = pd_ref[:, 0] / 10.0
    pd = jnp.where(pd < 0.0, 0.001, pd)
    col = lax.broadcasted_iota(jnp.int32, (H, W), 1).astype(jnp.float32)
    row = lax.broadcasted_iota(jnp.int32, (H, W), 0).astype(jnp.float32)
    uu = col - _U0
    vv = row - _V0
    r2c = (uu * uu + vv * vv)[None]
    r = gt / pd
    omr = 1.0 - r
    l1 = gt - pd
    dist = jnp.sqrt(r2c * (omr * omr) + l1 * l1 + _EPS)
    m = (gt > 0.0) & (gt <= 10.0) & bg
    mf = m.astype(jnp.float32)
    s_tot = jnp.sum(dist * mf)
    c_tot = jnp.sum(mf)
    out_ref[0, 0] = s_tot / jnp.maximum(c_tot, 1.0) / _FX


def kernel(rgb, depth_pred, depth_gt):
    out = pl.pallas_call(
        _vb_kernel,
        in_specs=[
            pl.BlockSpec((_B, 3, _H, _W), lambda: (0, 0, 0, 0)),
            pl.BlockSpec((_B, 1, _H, _W), lambda: (0, 0, 0, 0)),
            pl.BlockSpec((_B, 1, _H, _W), lambda: (0, 0, 0, 0)),
        ],
        out_specs=pl.BlockSpec((1, 1), lambda: (0, 0),
                               memory_space=pltpu.SMEM),
        out_shape=jax.ShapeDtypeStruct((1, 1), jnp.float32),
        scratch_shapes=[pltpu.VMEM((_B, _H, _W), jnp.int32)],
    )(rgb, depth_pred, depth_gt)
    return out[0, 0]
